# Initial kernel scaffold; baseline (speedup 1.0000x reference)
import functools

import jax
import jax.numpy as jnp
from jax import lax
from jax.experimental import pallas as pl
from jax.experimental.pallas import tpu as pltpu
from jax.experimental.pallas import tpu_sc as plsc

N = 10000
E = 320000
NC = 2
NS = 16
EB = 128
ESTEPS = 158
E_PAD = NS * ESTEPS * EB
HALF = ESTEPS // 2
ACC_ROWS = 10240
ROWS_PT = 640
OUT_PT = 625
SCALE_PAD = 640
DUMMY = N
IDROWS = 79
DEGP_LEN = IDROWS * EB

_mesh = plsc.VectorSubcoreMesh(
    core_axis_name="c", subcore_axis_name="s", num_cores=NC, num_subcores=NS)


@functools.partial(
    pl.kernel,
    out_type=(
        jax.ShapeDtypeStruct((NS, ESTEPS, EB), jnp.int32),
        jax.ShapeDtypeStruct((NC, NS, SCALE_PAD), jnp.float32),
        jax.ShapeDtypeStruct((NC, NS, SCALE_PAD), jnp.float32),
    ),
    mesh=_mesh,
    scratch_types=[
        pltpu.VMEM((HALF, EB), jnp.int32),
        pltpu.VMEM((HALF, EB), jnp.int32),
        pltpu.VMEM((HALF, EB), jnp.int32),
        pltpu.VMEM((IDROWS, EB), jnp.int32),
        pltpu.VMEM((DEGP_LEN,), jnp.float32),
        pltpu.VMEM((DEGP_LEN,), jnp.float32),
        pltpu.VMEM_SHARED((ACC_ROWS,), jnp.float32),
        pltpu.VMEM_SHARED((ACC_ROWS,), jnp.float32),
    ],
    name="sc_degrees",
)
def _deg_kernel(row3, col3, iden, col2_out, degp_out, degcp_out,
                rowc, colc, col2c, idn, degv, degcv, sdeg, sdegc):
    c = lax.axis_index("c")
    s = lax.axis_index("s")
    pltpu.sync_copy(row3.at[s, pl.ds(HALF * c, HALF)], rowc)
    pltpu.sync_copy(col3.at[s, pl.ds(HALF * c, HALF)], colc)
    pltpu.sync_copy(iden, idn)
    zeros16 = jnp.zeros((16,), jnp.float32)
    ones16 = jnp.ones((16,), jnp.float32)

    def zero_body(i, _):
        degv[pl.ds(16 * i, 16)] = zeros16
        degcv[pl.ds(16 * i, 16)] = zeros16
        return 0
    lax.fori_loop(0, DEGP_LEN // 16, zero_body, 0)
    pltpu.sync_copy(degv.at[pl.ds(0, ROWS_PT)], sdeg.at[pl.ds(ROWS_PT * s, ROWS_PT)])
    pltpu.sync_copy(degcv.at[pl.ds(0, ROWS_PT)], sdegc.at[pl.ds(ROWS_PT * s, ROWS_PT)])
    plsc.subcore_barrier()

    def body(j, _):
        for m in range(EB // 16):
            rv = rowc[j, pl.ds(16 * m, 16)]
            cv = colc[j, pl.ds(16 * m, 16)]
            msk = rv != cv
            plsc.addupdate_scatter(degv, [rv], ones16, mask=msk)
            plsc.addupdate_scatter(degcv, [cv], ones16, mask=msk)
            col2c[j, pl.ds(16 * m, 16)] = jnp.where(msk, cv, DUMMY)
        return 0
    lax.fori_loop(0, HALF, body, 0)
    pltpu.sync_copy(col2c, col2_out.at[s, pl.ds(HALF * c, HALF)])

    def addb(j, _):
        pltpu.sync_copy(degv.at[pl.ds(EB * j, EB)], sdeg.at[idn.at[j]], add=True)
        pltpu.sync_copy(degcv.at[pl.ds(EB * j, EB)], sdegc.at[idn.at[j]], add=True)
        return 0
    lax.fori_loop(0, IDROWS, addb, 0)
    plsc.subcore_barrier()
    pltpu.sync_copy(sdeg.at[pl.ds(ROWS_PT * s, ROWS_PT)], degv.at[pl.ds(0, ROWS_PT)])
    pltpu.sync_copy(sdegc.at[pl.ds(ROWS_PT * s, ROWS_PT)], degcv.at[pl.ds(0, ROWS_PT)])
    pltpu.sync_copy(degv.at[pl.ds(0, ROWS_PT)], degp_out.at[c, s])
    pltpu.sync_copy(degcv.at[pl.ds(0, ROWS_PT)], degcp_out.at[c, s])


def _make_S(nb, dc, name):
    nbpc = nb // NC

    @functools.partial(
        pl.kernel,
        out_type=jax.ShapeDtypeStruct((nb * N, dc), jnp.float32),
        mesh=_mesh,
        scratch_types=[
            pltpu.VMEM((ESTEPS, EB), jnp.int32),
            pltpu.VMEM((ESTEPS, EB), jnp.int32),
            pltpu.VMEM((ESTEPS, EB), jnp.int32),
            pltpu.VMEM((EB, dc), jnp.float32),
            pltpu.VMEM((EB, dc), jnp.float32),
            pltpu.VMEM((SCALE_PAD,), jnp.float32),
            pltpu.VMEM_SHARED((ACC_ROWS, dc), jnp.float32),
            pltpu.SemaphoreType.DMA,
            pltpu.SemaphoreType.DMA,
        ],
        name=name,
    )
    def S(v_hbm, row_hbm, col2_hbm, scale_hbm, out_hbm,
          rowc, rowc2, colc, buf0, buf1, scalec, acc, sem0, sem1):
        c = lax.axis_index("c")
        s = lax.axis_index("s")
        pltpu.sync_copy(row_hbm.at[s], rowc)
        pltpu.sync_copy(col2_hbm.at[s], colc)
        pltpu.sync_copy(scale_hbm.at[s], scalec)
        bufs = (buf0, buf1)
        sems = (sem0, sem1)
        zeros16 = jnp.zeros((16,), jnp.float32)
        for bl in range(nbpc):
            blk = c * nbpc + bl
            off = blk * N

            def offs_body(j, _):
                for m in range(EB // 16):
                    rowc2[j, pl.ds(16 * m, 16)] = rowc[j, pl.ds(16 * m, 16)] + off
                return 0
            lax.fori_loop(0, ESTEPS, offs_body, 0)

            def zero_body(i, _):
                for m in range(dc // 16):
                    buf0[i, pl.ds(16 * m, 16)] = zeros16
                return 0
            lax.fori_loop(0, EB, zero_body, 0)
            for k in range(ROWS_PT // EB):
                pltpu.sync_copy(buf0, acc.at[pl.ds(ROWS_PT * s + EB * k, EB)])
            plsc.subcore_barrier()

            pltpu.async_copy(v_hbm.at[rowc2.at[0]], buf0, sem0)
            pltpu.async_copy(v_hbm.at[rowc2.at[1]], buf1, sem1)

            def step(jj, _):
                for b in range(2):
                    j = 2 * jj + b
                    pltpu.make_async_copy(v_hbm.at[rowc2.at[0]], bufs[b], sems[b]).wait()
                    pltpu.sync_copy(bufs[b], acc.at[colc.at[j]], add=True)
                    pltpu.async_copy(v_hbm.at[rowc2.at[j + 2]], bufs[b], sems[b])
                return 0
            lax.fori_loop(0, ESTEPS // 2 - 1, step, 0)
            for b in range(2):
                j = ESTEPS - 2 + b
                pltpu.make_async_copy(v_hbm.at[rowc2.at[0]], bufs[b], sems[b]).wait()
                pltpu.sync_copy(bufs[b], acc.at[colc.at[j]], add=True)
            plsc.subcore_barrier()

            for k5 in range(OUT_PT // 125):
                base = OUT_PT * s + 125 * k5
                pltpu.sync_copy(acc.at[pl.ds(base, 125)], buf0.at[pl.ds(0, 125)])

                def scale_body(r, _):
                    sc = scalec[125 * k5 + r]
                    for m in range(dc // 16):
                        buf0[r, pl.ds(16 * m, 16)] = buf0[r, pl.ds(16 * m, 16)] * sc
                    return 0
                lax.fori_loop(0, 125, scale_body, 0)
                pltpu.sync_copy(buf0.at[pl.ds(0, 125)],
                                out_hbm.at[pl.ds(off + base, 125)])
            if nbpc > 1:
                plsc.subcore_barrier()
    return S


_S64 = _make_S(2, 64, "sc_S_nb2_dc64")
_S128 = _make_S(2, 128, "sc_S_nb2_dc128")
_S128x2 = _make_S(4, 128, "sc_S_nb4_dc128")
_S32 = _make_S(2, 32, "sc_S_nb2_dc32")


def _scal_body(degp_ref, degcp_ref, dinv_ref, dinv2_ref, g_ref,
               dinvc_ref, dinvc2_ref):
    deg = degp_ref[0] + degp_ref[1]
    degc = degcp_ref[0] + degcp_ref[1] + 1.0
    pos = deg > 0.0
    safe = jnp.where(pos, deg, 1.0)
    dinv = jnp.where(pos, lax.rsqrt(safe), 0.0)
    dinv_ref[...] = dinv
    dinv2_ref[...] = dinv * dinv
    g_ref[...] = jnp.where(pos, jnp.sqrt(safe), 0.0)
    dinvc_ref[...] = lax.rsqrt(degc)
    dinvc2_ref[...] = 1.0 / degc


_scal = pl.pallas_call(
    _scal_body,
    out_shape=[jax.ShapeDtypeStruct((NS, SCALE_PAD), jnp.float32)] * 5,
)


_BM = 1000


def _z0_body(x_ref, dinv_ref, z_ref):
    z = x_ref[...] * dinv_ref[...]
    z_ref[0] = z[:, :64]
    z_ref[1] = z[:, 64:]


_z0 = pl.pallas_call(
    _z0_body,
    grid=(N // _BM,),
    in_specs=[
        pl.BlockSpec((_BM, 128), lambda i: (i, 0)),
        pl.BlockSpec((_BM, 1), lambda i: (i, 0)),
    ],
    out_specs=pl.BlockSpec((2, _BM, 64), lambda i: (0, i, 0)),
    out_shape=jax.ShapeDtypeStruct((2, N, 64), jnp.float32),
)


def _make_mm3(nbk, dck, M, with_z, name):
    K = nbk * dck
    grid = (N // _BM, M // 128)

    def compute(h_ref, p_ref, q_ref, g_ref, w0_ref, w1_ref, w2_ref, b_ref):
        acc = jnp.dot(h_ref[...], w0_ref[...], preferred_element_type=jnp.float32)
        gv = g_ref[...]
        for bb in range(nbk):
            acc += jnp.dot(p_ref[bb] * gv, w1_ref[bb],
                           preferred_element_type=jnp.float32)
            acc += jnp.dot(q_ref[bb], w2_ref[bb],
                           preferred_element_type=jnp.float32)
        return jnp.maximum(acc + b_ref[...], 0.0)

    def bodyz(h_ref, p_ref, q_ref, g_ref, dinv_ref, w0_ref, w1_ref, w2_ref,
              b_ref, h_out_ref, z_out_ref):
        out = compute(h_ref, p_ref, q_ref, g_ref, w0_ref, w1_ref, w2_ref, b_ref)
        h_out_ref[...] = out
        z_out_ref[0] = out * dinv_ref[...]

    def bodynz(h_ref, p_ref, q_ref, g_ref, dinv_ref, w0_ref, w1_ref, w2_ref,
               b_ref, h_out_ref):
        h_out_ref[...] = compute(h_ref, p_ref, q_ref, g_ref, w0_ref, w1_ref,
                                 w2_ref, b_ref)

    in_specs = [
        pl.BlockSpec((_BM, K), lambda i, j: (i, 0)),
        pl.BlockSpec((nbk, _BM, dck), lambda i, j: (0, i, 0)),
        pl.BlockSpec((nbk, _BM, dck), lambda i, j: (0, i, 0)),
        pl.BlockSpec((_BM, 1), lambda i, j: (i, 0)),
        pl.BlockSpec((_BM, 1), lambda i, j: (i, 0)),
        pl.BlockSpec((K, 128), lambda i, j: (0, j)),
        pl.BlockSpec((nbk, dck, 128), lambda i, j: (0, 0, j)),
        pl.BlockSpec((nbk, dck, 128), lambda i, j: (0, 0, j)),
        pl.BlockSpec((1, 128), lambda i, j: (0, j)),
    ]
    out_specs = [pl.BlockSpec((_BM, 128), lambda i, j: (i, j))]
    out_shape = [jax.ShapeDtypeStruct((N, M), jnp.float32)]
    if with_z:
        out_specs.append(pl.BlockSpec((1, _BM, 128), lambda i, j: (j, i, 0)))
        out_shape.append(jax.ShapeDtypeStruct((M // 128, N, 128), jnp.float32))
    return pl.pallas_call(
        bodyz if with_z else bodynz,
        grid=grid,
        in_specs=in_specs,
        out_specs=out_specs,
        out_shape=out_shape,
        name=name,
    )


_mm3_1 = _make_mm3(2, 64, 256, True, "tc_cheb1")
_mm3_2 = _make_mm3(2, 128, 512, True, "tc_cheb2")
_mm3_3 = _make_mm3(4, 128, 1024, False, "tc_cheb3")


def _gcnA_body(h_ref, w_ref, dinvc_ref, hw_ref, u_ref):
    hw = jnp.dot(h_ref[...], w_ref[...], preferred_element_type=jnp.float32)
    hw_ref[...] = hw
    u = hw * dinvc_ref[...]
    u_ref[0] = u[:, :32]
    u_ref[1] = u[:, 32:]


_gcnA = pl.pallas_call(
    _gcnA_body,
    grid=(N // _BM,),
    in_specs=[
        pl.BlockSpec((_BM, 1024), lambda i: (i, 0)),
        pl.BlockSpec((1024, 64), lambda i: (0, 0)),
        pl.BlockSpec((_BM, 1), lambda i: (i, 0)),
    ],
    out_specs=[
        pl.BlockSpec((_BM, 64), lambda i: (i, 0)),
        pl.BlockSpec((2, _BM, 32), lambda i: (0, i, 0)),
    ],
    out_shape=[
        jax.ShapeDtypeStruct((N, 64), jnp.float32),
        jax.ShapeDtypeStruct((2, N, 32), jnp.float32),
    ],
)


def _gcnB_body(pc_ref, hw_ref, dinvc2_ref, bc_ref, logits_ref, probs_ref):
    lt = jnp.concatenate([pc_ref[0], pc_ref[1]], axis=1)
    lt = lt + dinvc2_ref[...] * hw_ref[...] + bc_ref[...]
    lt = lt[:, :40]
    logits_ref[...] = lt
    m = jnp.max(lt, axis=1, keepdims=True)
    e = jnp.exp(lt - m)
    probs_ref[...] = e / jnp.sum(e, axis=1, keepdims=True)


_gcnB = pl.pallas_call(
    _gcnB_body,
    grid=(N // _BM,),
    in_specs=[
        pl.BlockSpec((2, _BM, 32), lambda i: (0, i, 0)),
        pl.BlockSpec((_BM, 64), lambda i: (i, 0)),
        pl.BlockSpec((_BM, 1), lambda i: (i, 0)),
        pl.BlockSpec((1, 64), lambda i: (0, 0)),
    ],
    out_specs=[
        pl.BlockSpec((_BM, 40), lambda i: (i, 0)),
        pl.BlockSpec((_BM, 40), lambda i: (i, 0)),
    ],
    out_shape=[
        jax.ShapeDtypeStruct((N, 40), jnp.float32),
        jax.ShapeDtypeStruct((N, 40), jnp.float32),
    ],
)


def _flat(a):
    return a.reshape(-1)[:N]


def _coln(a):
    return _flat(a).reshape(N, 1)


def _tilel(a):
    return jnp.pad(_flat(a).reshape(NS, OUT_PT),
                   ((0, 0), (0, SCALE_PAD - OUT_PT)))


def _padW(W, Kp, Mp):
    return jnp.pad(W, ((0, Kp - W.shape[0]), (0, Mp - W.shape[1])))


def kernel(x, edge_index, W1_0, W1_1, W1_2, b1, W2_0, W2_1, W2_2, b2,
           W3_0, W3_1, W3_2, b3, Wc, bc):
    row = edge_index[0]
    col = edge_index[1]
    padn = E_PAD - E
    rowp = jnp.concatenate([row, jnp.zeros((padn,), row.dtype)])
    colp = jnp.concatenate([col, jnp.zeros((padn,), col.dtype)])
    row3 = rowp.reshape(NS, ESTEPS, EB)
    col3 = colp.reshape(NS, ESTEPS, EB)
    iden = jnp.arange(DEGP_LEN, dtype=jnp.int32).reshape(IDROWS, EB)

    col2, degp, degcp = _deg_kernel(row3, col3, iden)
    dinv6, dinv26, g6, dinvc6, dinvc26 = _scal(degp, degcp)
    dinv_c = _coln(dinv6)
    g_c = _coln(g6)
    dinvc_c = _coln(dinvc6)
    dinvc2_c = _coln(dinvc26)
    dinv_t = _tilel(dinv6)
    dinv2_t = _tilel(dinv26)
    dinvc_t = _tilel(dinvc6)

    dims = ((128, 256, 64), (256, 512, 128), (512, 1024, 128))
    Ws = ((W1_0, W1_1, W1_2, b1), (W2_0, W2_1, W2_2, b2), (W3_0, W3_1, W3_2, b3))
    mms = (_mm3_1, _mm3_2, _mm3_3)
    Ss = (_S64, _S128, _S128x2)

    z = _z0(x, dinv_c).reshape(2 * N, 64)
    h = x
    for li in range(3):
        Kp, Mp, dck = dims[li]
        W0, W1, W2, b = Ws[li]
        nbk = Kp // dck
        w0 = _padW(W0 - W2, Kp, Mp)
        w1 = _padW(-W1, Kp, Mp).reshape(nbk, dck, Mp)
        w2 = _padW(2.0 * W2, Kp, Mp).reshape(nbk, dck, Mp)
        bp = jnp.pad(b, (0, Mp - b.shape[0])).reshape(1, Mp)
        p = Ss[li](z, row3, col2, dinv2_t)
        q2 = Ss[li](p, row3, col2, dinv_t)
        p = p.reshape(nbk, N, dck)
        q2 = q2.reshape(nbk, N, dck)
        if li < 2:
            h, z = mms[li](h, p, q2, g_c, dinv_c, w0, w1, w2, bp)
            z = z.reshape(-1, 128)
        else:
            (h,) = mms[li](h, p, q2, g_c, dinv_c, w0, w1, w2, bp)
    wc = _padW(Wc, 1024, 64)
    bcp = jnp.pad(bc, (0, 24)).reshape(1, 64)
    hw, u = _gcnA(h, wc, dinvc_c)
    pc = _S32(u.reshape(2 * N, 32), row3, col2, dinvc_t)
    logits, probs = _gcnB(pc.reshape(2, N, 32), hw, dinvc2_c, bcp)
    return logits, probs


# speedup vs baseline: 4.6972x; 4.6972x over previous
import functools

import jax
import jax.numpy as jnp
from jax import lax
from jax.experimental import pallas as pl
from jax.experimental.pallas import tpu as pltpu
from jax.experimental.pallas import tpu_sc as plsc

N = 10000
E = 320000
NC = 2
NS = 16
EB = 128
ESTEPS = 160
G = 16
NG = ESTEPS // G
E_PAD = NS * ESTEPS * EB
HALF = ESTEPS // 2
NROWS = 10240
ACC_ROWS = 10240
ROWS_PT = 640
SCALE_PAD = 656
DUMMY = N
IDROWS = 80
DEGP_LEN = IDROWS * EB

_mesh = plsc.VectorSubcoreMesh(
    core_axis_name="c", subcore_axis_name="s", num_cores=NC, num_subcores=NS)


@functools.partial(
    pl.kernel,
    out_type=(
        jax.ShapeDtypeStruct((NC * NS, HALF, EB), jnp.int32),
        jax.ShapeDtypeStruct((NC, 1, ACC_ROWS), jnp.float32),
        jax.ShapeDtypeStruct((NC, 1, ACC_ROWS), jnp.float32),
    ),
    mesh=_mesh,
    scratch_types=[
        pltpu.VMEM((HALF, EB), jnp.int32),
        pltpu.VMEM((HALF, EB), jnp.int32),
        pltpu.VMEM((HALF, EB), jnp.int32),
        pltpu.VMEM((IDROWS, EB), jnp.int32),
        pltpu.VMEM((DEGP_LEN,), jnp.float32),
        pltpu.VMEM((DEGP_LEN,), jnp.float32),
        pltpu.VMEM_SHARED((ACC_ROWS,), jnp.float32),
        pltpu.VMEM_SHARED((ACC_ROWS,), jnp.float32),
    ],
    compiler_params=pltpu.CompilerParams(needs_layout_passes=False),
    name="sc_degrees",
)
def _deg_kernel(row32, col32, iden, col2_out, degp_out, degcp_out,
                rowc, colc, col2c, idn, degv, degcv, sdeg, sdegc):
    c = lax.axis_index("c")
    s = lax.axis_index("s")
    w = 2 * s + c
    pltpu.sync_copy(row32.at[w], rowc)
    pltpu.sync_copy(col32.at[w], colc)
    pltpu.sync_copy(iden, idn)
    zeros16 = jnp.zeros((16,), jnp.float32)
    ones16 = jnp.ones((16,), jnp.float32)

    def zero_body(i, _):
        degv[pl.ds(16 * i, 16)] = zeros16
        degcv[pl.ds(16 * i, 16)] = zeros16
        return 0
    lax.fori_loop(0, DEGP_LEN // 16, zero_body, 0)
    pltpu.sync_copy(degv.at[pl.ds(0, ROWS_PT)], sdeg.at[pl.ds(ROWS_PT * s, ROWS_PT)])
    pltpu.sync_copy(degcv.at[pl.ds(0, ROWS_PT)], sdegc.at[pl.ds(ROWS_PT * s, ROWS_PT)])
    plsc.subcore_barrier()

    def body(j, _):
        for m in range(EB // 16):
            rv = rowc[j, pl.ds(16 * m, 16)]
            cv = colc[j, pl.ds(16 * m, 16)]
            msk = rv != cv
            plsc.addupdate_scatter(degv, [rv], ones16, mask=msk)
            plsc.addupdate_scatter(degcv, [cv], ones16, mask=msk)
            col2c[j, pl.ds(16 * m, 16)] = jnp.where(msk, cv, DUMMY)
        return 0
    lax.fori_loop(0, HALF, body, 0)
    pltpu.sync_copy(col2c, col2_out.at[w])

    def addb(j, _):
        pltpu.sync_copy(degv.at[pl.ds(EB * j, EB)], sdeg.at[idn.at[j]], add=True)
        pltpu.sync_copy(degcv.at[pl.ds(EB * j, EB)], sdegc.at[idn.at[j]], add=True)
        return 0
    lax.fori_loop(0, IDROWS, addb, 0)
    plsc.subcore_barrier()
    pltpu.sync_copy(sdeg.at[pl.ds(ROWS_PT * s, ROWS_PT)],
                    degp_out.at[c, 0, pl.ds(ROWS_PT * s, ROWS_PT)])
    pltpu.sync_copy(sdegc.at[pl.ds(ROWS_PT * s, ROWS_PT)],
                    degcp_out.at[c, 0, pl.ds(ROWS_PT * s, ROWS_PT)])


def _make_S(nb, name):
    dc = 128
    nbpc = max(nb // NC, 1)

    @functools.partial(
        pl.kernel,
        out_type=jax.ShapeDtypeStruct((nb * NROWS, dc), jnp.float32),
        mesh=_mesh,
        scratch_types=[
            pltpu.VMEM((2, 2, G, EB), jnp.int32),
            pltpu.VMEM((EB, dc), jnp.float32),
            pltpu.VMEM((EB, dc), jnp.float32),
            pltpu.VMEM((1, SCALE_PAD), jnp.float32),
            pltpu.VMEM_SHARED((ACC_ROWS, dc), jnp.float32),
            pltpu.SemaphoreType.DMA,
            pltpu.SemaphoreType.DMA,
            pltpu.SemaphoreType.DMA,
            pltpu.SemaphoreType.DMA,
        ],
        name=name,
    )
    def S(v_hbm, ec_hbm, scale_hbm, out_hbm,
          idxb, buf0, buf1, scalec, acc, semi0, semi1, sem0, sem1):
        c = lax.axis_index("c")
        s = lax.axis_index("s")
        pltpu.sync_copy(scale_hbm.at[s], scalec)
        bufs = (buf0, buf1)
        semd = (sem0, sem1)
        semi = (semi0, semi1)
        zeros16 = jnp.zeros((16,), jnp.float32)

        def run_block(blk, is_last):
            off = blk * NROWS

            def zero_body(i, _):
                for m in range(dc // 16):
                    buf0[i, pl.ds(16 * m, 16)] = zeros16
                return 0
            lax.fori_loop(0, EB, zero_body, 0)
            for k in range(ROWS_PT // EB):
                pltpu.sync_copy(buf0, acc.at[pl.ds(ROWS_PT * s + EB * k, EB)])
            plsc.subcore_barrier()

            pltpu.async_copy(ec_hbm.at[s, 0], idxb.at[0], semi[0])

            def group_pair(gg, _):
                for p in range(2):
                    g = 2 * gg + p
                    pltpu.async_copy(ec_hbm.at[s, g + 1], idxb.at[1 - p],
                                     semi[1 - p])
                    pltpu.make_async_copy(ec_hbm.at[s, 0], idxb.at[p],
                                          semi[p]).wait()

                    def adj(t, _):
                        for m in range(EB // 16):
                            sl = pl.ds(16 * m, 16)
                            idxb[p, 0, t, sl] = idxb[p, 0, t, sl] + off
                        return 0
                    lax.fori_loop(0, G, adj, 0)
                    pltpu.async_copy(v_hbm.at[idxb.at[p, 0, 0]], buf0, sem0)
                    pltpu.async_copy(v_hbm.at[idxb.at[p, 0, 1]], buf1, sem1)
                    for t in range(G):
                        b = t % 2
                        pltpu.make_async_copy(v_hbm.at[idxb.at[p, 0, 0]],
                                              bufs[b], semd[b]).wait()
                        pltpu.sync_copy(bufs[b], acc.at[idxb.at[p, 1, t]],
                                        add=True)
                        if t < G - 2:
                            pltpu.async_copy(v_hbm.at[idxb.at[p, 0, t + 2]],
                                             bufs[b], semd[b])
                return 0
            lax.fori_loop(0, NG // 2, group_pair, 0)
            pltpu.make_async_copy(ec_hbm.at[s, 0], idxb.at[0], semi[0]).wait()
            plsc.subcore_barrier()

            for k5 in range(ROWS_PT // EB):
                base = ROWS_PT * s + EB * k5
                pltpu.sync_copy(acc.at[pl.ds(base, EB)], buf0)

                def scale_body(r, _):
                    sc = scalec[0, pl.ds(EB * k5 + r, 16)][0]
                    for m in range(dc // 16):
                        buf0[r, pl.ds(16 * m, 16)] = buf0[r, pl.ds(16 * m, 16)] * sc
                    return 0
                lax.fori_loop(0, EB, scale_body, 0)
                pltpu.sync_copy(buf0, out_hbm.at[pl.ds(off + base, EB)])
            if not is_last:
                plsc.subcore_barrier()

        if nb == 1:
            pl.when(c == 0)(lambda: run_block(0, True))
        else:
            for bl in range(nbpc):
                run_block(c * nbpc + bl, bl == nbpc - 1)
    return S


_S1 = _make_S(1, "sc_S_nb1")
_S2 = _make_S(2, "sc_S_nb2")
_S4 = _make_S(4, "sc_S_nb4")


def _scal_body(degp_ref, degcp_ref, dinv_ref, dinv2_ref, g_ref,
               dinvc_ref, dinvc2_ref):
    deg = degp_ref[0] + degp_ref[1]
    degc = degcp_ref[0] + degcp_ref[1] + 1.0
    pos = deg > 0.0
    safe = jnp.where(pos, deg, 1.0)
    dinv = jnp.where(pos, lax.rsqrt(safe), 0.0)
    dinv_ref[...] = dinv
    dinv2_ref[...] = dinv * dinv
    g_ref[...] = jnp.where(pos, jnp.sqrt(safe), 0.0)
    dinvc_ref[...] = lax.rsqrt(degc)
    dinvc2_ref[...] = 1.0 / degc


_scal = pl.pallas_call(
    _scal_body,
    out_shape=[jax.ShapeDtypeStruct((80, 128), jnp.float32)] * 5,
)


_BM = 1000


def _z0_body(x_ref, dinv_ref, z_ref):
    z_ref[...] = x_ref[...] * dinv_ref[...]


_z0 = pl.pallas_call(
    _z0_body,
    grid=(N // _BM,),
    in_specs=[
        pl.BlockSpec((_BM, 128), lambda i: (i, 0)),
        pl.BlockSpec((_BM, 1), lambda i: (i, 0)),
    ],
    out_specs=pl.BlockSpec((_BM, 128), lambda i: (i, 0)),
    out_shape=jax.ShapeDtypeStruct((NROWS, 128), jnp.float32),
)


def _make_mm3(nbk, dck, M, with_z, name):
    K = nbk * dck
    grid = (N // _BM, M // 128)

    def compute(h_ref, p_ref, q_ref, g_ref, w0_ref, w1_ref, w2_ref, b_ref):
        acc = jnp.dot(h_ref[...], w0_ref[...], preferred_element_type=jnp.float32)
        gv = g_ref[...]
        for bb in range(nbk):
            acc += jnp.dot(p_ref[bb] * gv, w1_ref[bb],
                           preferred_element_type=jnp.float32)
            acc += jnp.dot(q_ref[bb], w2_ref[bb],
                           preferred_element_type=jnp.float32)
        return jnp.maximum(acc + b_ref[...], 0.0)

    def bodyz(h_ref, p_ref, q_ref, g_ref, dinv_ref, w0_ref, w1_ref, w2_ref,
              b_ref, h_out_ref, z_out_ref):
        out = compute(h_ref, p_ref, q_ref, g_ref, w0_ref, w1_ref, w2_ref, b_ref)
        h_out_ref[...] = out
        z_out_ref[0] = out * dinv_ref[...]

    def bodynz(h_ref, p_ref, q_ref, g_ref, dinv_ref, w0_ref, w1_ref, w2_ref,
               b_ref, h_out_ref):
        h_out_ref[...] = compute(h_ref, p_ref, q_ref, g_ref, w0_ref, w1_ref,
                                 w2_ref, b_ref)

    in_specs = [
        pl.BlockSpec((_BM, K), lambda i, j: (i, 0)),
        pl.BlockSpec((nbk, _BM, dck), lambda i, j: (0, i, 0)),
        pl.BlockSpec((nbk, _BM, dck), lambda i, j: (0, i, 0)),
        pl.BlockSpec((_BM, 1), lambda i, j: (i, 0)),
        pl.BlockSpec((_BM, 1), lambda i, j: (i, 0)),
        pl.BlockSpec((K, 128), lambda i, j: (0, j)),
        pl.BlockSpec((nbk, dck, 128), lambda i, j: (0, 0, j)),
        pl.BlockSpec((nbk, dck, 128), lambda i, j: (0, 0, j)),
        pl.BlockSpec((1, 128), lambda i, j: (0, j)),
    ]
    out_specs = [pl.BlockSpec((_BM, 128), lambda i, j: (i, j))]
    out_shape = [jax.ShapeDtypeStruct((N, M), jnp.float32)]
    if with_z:
        out_specs.append(pl.BlockSpec((1, _BM, 128), lambda i, j: (j, i, 0)))
        out_shape.append(jax.ShapeDtypeStruct((M // 128, NROWS, 128), jnp.float32))
    return pl.pallas_call(
        bodyz if with_z else bodynz,
        grid=grid,
        in_specs=in_specs,
        out_specs=out_specs,
        out_shape=out_shape,
        name=name,
    )


_mm3_1 = _make_mm3(1, 128, 256, True, "tc_cheb1")
_mm3_2 = _make_mm3(2, 128, 512, True, "tc_cheb2")
_mm3_3 = _make_mm3(4, 128, 1024, False, "tc_cheb3")


def _gcnA_body(h_ref, w_ref, dinvc_ref, hw_ref, u_ref):
    hw = jnp.dot(h_ref[...], w_ref[...], preferred_element_type=jnp.float32)
    hw_ref[...] = hw
    u = hw * dinvc_ref[...]
    u_ref[...] = jnp.concatenate(
        [u, jnp.zeros((u.shape[0], 64), jnp.float32)], axis=1)


_gcnA = pl.pallas_call(
    _gcnA_body,
    grid=(N // _BM,),
    in_specs=[
        pl.BlockSpec((_BM, 1024), lambda i: (i, 0)),
        pl.BlockSpec((1024, 64), lambda i: (0, 0)),
        pl.BlockSpec((_BM, 1), lambda i: (i, 0)),
    ],
    out_specs=[
        pl.BlockSpec((_BM, 64), lambda i: (i, 0)),
        pl.BlockSpec((_BM, 128), lambda i: (i, 0)),
    ],
    out_shape=[
        jax.ShapeDtypeStruct((N, 64), jnp.float32),
        jax.ShapeDtypeStruct((NROWS, 128), jnp.float32),
    ],
)


def _gcnB_body(pc_ref, hw_ref, dinvc2_ref, bc_ref, logits_ref, probs_ref):
    lt = pc_ref[...][:, :64] + dinvc2_ref[...] * hw_ref[...] + bc_ref[...]
    lt = lt[:, :40]
    logits_ref[...] = lt
    m = jnp.max(lt, axis=1, keepdims=True)
    e = jnp.exp(lt - m)
    probs_ref[...] = e / jnp.sum(e, axis=1, keepdims=True)


_gcnB = pl.pallas_call(
    _gcnB_body,
    grid=(N // _BM,),
    in_specs=[
        pl.BlockSpec((_BM, 128), lambda i: (i, 0)),
        pl.BlockSpec((_BM, 64), lambda i: (i, 0)),
        pl.BlockSpec((_BM, 1), lambda i: (i, 0)),
        pl.BlockSpec((1, 64), lambda i: (0, 0)),
    ],
    out_specs=[
        pl.BlockSpec((_BM, 40), lambda i: (i, 0)),
        pl.BlockSpec((_BM, 40), lambda i: (i, 0)),
    ],
    out_shape=[
        jax.ShapeDtypeStruct((N, 40), jnp.float32),
        jax.ShapeDtypeStruct((N, 40), jnp.float32),
    ],
)


def _flat(a):
    return a.reshape(-1)[:N]


def _coln(a):
    return _flat(a).reshape(N, 1)


def _tilel(a):
    f = a.reshape(NS, 1, ROWS_PT)
    return jnp.pad(f, ((0, 0), (0, 0), (0, SCALE_PAD - ROWS_PT)))


def _padW(W, Kp, Mp):
    return jnp.pad(W, ((0, Kp - W.shape[0]), (0, Mp - W.shape[1])))


def kernel(x, edge_index, W1_0, W1_1, W1_2, b1, W2_0, W2_1, W2_2, b2,
           W3_0, W3_1, W3_2, b3, Wc, bc):
    row = edge_index[0]
    col = edge_index[1]
    padn = E_PAD - E
    rowp = jnp.concatenate([row, jnp.zeros((padn,), row.dtype)])
    colp = jnp.concatenate([col, jnp.zeros((padn,), col.dtype)])
    row32 = rowp.reshape(NC * NS, HALF, EB)
    col32 = colp.reshape(NC * NS, HALF, EB)
    iden = jnp.arange(DEGP_LEN, dtype=jnp.int32).reshape(IDROWS, EB)

    col2w, degp, degcp = _deg_kernel(row32, col32, iden)
    degp = degp.reshape(NC, 80, 128)
    degcp = degcp.reshape(NC, 80, 128)
    rowg = rowp.reshape(NS, NG, G, EB)
    colg = col2w.reshape(NS, NG, G, EB)
    ec = jnp.pad(jnp.stack([rowg, colg], axis=2),
                 ((0, 0), (0, 2), (0, 0), (0, 0), (0, 0)))
    dinv6, dinv26, g6, dinvc6, dinvc26 = _scal(degp, degcp)
    dinv_c = _coln(dinv6)
    g_c = _coln(g6)
    dinvc_c = _coln(dinvc6)
    dinvc2_c = _coln(dinvc26)
    dinv_t = _tilel(dinv6)
    dinv2_t = _tilel(dinv26)
    dinvc_t = _tilel(dinvc6)

    dims = ((128, 256, 128), (256, 512, 128), (512, 1024, 128))
    Ws = ((W1_0, W1_1, W1_2, b1), (W2_0, W2_1, W2_2, b2), (W3_0, W3_1, W3_2, b3))
    mms = (_mm3_1, _mm3_2, _mm3_3)
    Ss = (_S1, _S2, _S4)

    z = _z0(x, dinv_c)
    h = x
    for li in range(3):
        Kp, Mp, dck = dims[li]
        W0, W1, W2, b = Ws[li]
        nbk = Kp // dck
        w0 = _padW(W0 - W2, Kp, Mp)
        w1 = _padW(-W1, Kp, Mp).reshape(nbk, dck, Mp)
        w2 = _padW(2.0 * W2, Kp, Mp).reshape(nbk, dck, Mp)
        bp = jnp.pad(b, (0, Mp - b.shape[0])).reshape(1, Mp)
        p = Ss[li](z, ec, dinv2_t)
        q2 = Ss[li](p, ec, dinv_t)
        p = p.reshape(nbk, NROWS, dck)
        q2 = q2.reshape(nbk, NROWS, dck)
        if li < 2:
            h, z = mms[li](h, p, q2, g_c, dinv_c, w0, w1, w2, bp)
            z = z.reshape(-1, 128)
        else:
            (h,) = mms[li](h, p, q2, g_c, dinv_c, w0, w1, w2, bp)
    wc = _padW(Wc, 1024, 64)
    bcp = jnp.pad(bc, (0, 24)).reshape(1, 64)
    hw, u = _gcnA(h, wc, dinvc_c)
    pc = _S1(u, ec, dinvc_t)
    logits, probs = _gcnB(pc, hw, dinvc2_c, bcp)
    return logits, probs
